# baseline (speedup 1.0000x reference)
import functools

import jax
import jax.numpy as jnp
from jax import lax
from jax.experimental import pallas as pl
from jax.experimental.pallas import tpu as pltpu
from jax.experimental.pallas import tpu_sc as plsc

ED = 64
NUM_OBJS = 40
NUM_PREDS = 26
O = 10000
T = 20000
DIN = 2 * ED
H = 4 * ED

BO = 1000
BT = 1024

NC, NS = 2, 16
NW = NC * NS
TP = 20480
OP = 10240
DUMP = O
GCW = 64
GNB = 6
GCH = TP // NW // GCW
GCHP = 16
SCH = TP // NS // 128
SCHP = 16
WB = OP // NS

@functools.lru_cache(maxsize=None)
def _get_sc_gather(width):
    mesh = plsc.VectorSubcoreMesh(core_axis_name="c", subcore_axis_name="s")

    @functools.partial(
        pl.kernel, mesh=mesh,
        out_type=[jax.ShapeDtypeStruct((TP, width), jnp.float32),
                  jax.ShapeDtypeStruct((TP, width), jnp.float32)],
        scratch_types=(
            [pltpu.VMEM((GCHP, GCW), jnp.int32),
             pltpu.VMEM((GCHP, GCW), jnp.int32)]
            + [pltpu.VMEM((GCW, width), jnp.float32)] * GNB
            + [pltpu.SemaphoreType.DMA] * (2 * GNB)),
    )
    def sc_gather(a_hbm, c_hbm, sidx_hbm, oidx_hbm, as_out, cs_out,
                  sidx_v, oidx_v, *bufs_sems):
        wid = lax.axis_index("s") * NC + lax.axis_index("c")
        pltpu.sync_copy(sidx_hbm.at[pl.ds(wid * GCHP, GCHP)], sidx_v)
        pltpu.sync_copy(oidx_hbm.at[pl.ds(wid * GCHP, GCHP)], oidx_v)
        bufs = bufs_sems[:GNB]
        gsems = bufs_sems[GNB:2 * GNB]
        wsems = bufs_sems[2 * GNB:]
        nbuf = GNB
        ntask = 2 * GCH

        def issue(t, buf, sem):
            j = t // 2
            if t % 2 == 0:
                return pltpu.async_copy(a_hbm.at[sidx_v.at[j]], buf, sem)
            return pltpu.async_copy(c_hbm.at[oidx_v.at[j]], buf, sem)

        def out_dst(t):
            base = wid * (GCH * GCW) + (t // 2) * GCW
            dst = as_out if t % 2 == 0 else cs_out
            return dst.at[pl.ds(base, GCW)]

        gd = [issue(b, bufs[b], gsems[b]) for b in range(nbuf)]
        wd = [None] * nbuf
        for t in range(ntask):
            b = t % nbuf
            gd[b].wait()
            wd[b] = pltpu.async_copy(bufs[b], out_dst(t), wsems[b])
            if t + nbuf < ntask:
                wd[b].wait()
                gd[b] = issue(t + nbuf, bufs[b], gsems[b])
        for b in range(nbuf):
            if wd[b] is not None:
                wd[b].wait()

    return sc_gather


def _sc_gather(a, c, sidx, oidx):
    return _get_sc_gather(a.shape[1])(a, c, sidx, oidx)


@functools.lru_cache(maxsize=None)
def _get_sc_scatter():
    mesh = plsc.VectorSubcoreMesh(core_axis_name="c", subcore_axis_name="s")

    @functools.partial(
        pl.kernel, mesh=mesh,
        out_type=jax.ShapeDtypeStruct((OP, H), jnp.float32),
        scratch_types=[pltpu.VMEM((SCHP, 128), jnp.int32),
                       pltpu.VMEM((SCHP, 128), jnp.int32),
                       pltpu.VMEM((128, 128), jnp.float32),
                       pltpu.VMEM((128, 128), jnp.float32),
                       pltpu.SemaphoreType.DMA,
                       pltpu.SemaphoreType.DMA,
                       pltpu.VMEM_SHARED((OP, 128), jnp.float32)],
    )
    def sc_scatter(ns_hbm, no_hbm, sidx_hbm, oidx_hbm, pooled_out,
                   sidx_v, oidx_v, buf0_v, buf1_v, l0, l1, shared):
        c = lax.axis_index("c")
        s16 = lax.axis_index("s")
        rows0 = s16 * (SCH * 128)
        pltpu.sync_copy(sidx_hbm.at[pl.ds(s16 * SCHP, SCHP)], sidx_v)
        pltpu.sync_copy(oidx_hbm.at[pl.ds(s16 * SCHP, SCHP)], oidx_v)

        zv = jnp.zeros((16,), jnp.float32)

        def zfill(r, carry):
            for k in range(8):
                buf0_v[r, pl.ds(k * 16, 16)] = zv
            return carry

        lax.fori_loop(0, 128, zfill, 0)
        for w in range(WB // 128):
            pltpu.sync_copy(buf0_v,
                            shared.at[pl.ds(s16 * WB + w * 128, 128)])
        plsc.subcore_barrier()

        bufs = (buf0_v, buf1_v)
        lsems = (l0, l1)
        ntask = 2 * SCH

        def issue(t, buf, sem):
            r = rows0 + (t // 2) * 128
            src = ns_hbm if t % 2 == 0 else no_hbm
            return pltpu.async_copy(
                src.at[pl.ds(r, 128), pl.ds(c * 128, 128)], buf, sem)

        def idx_row(t):
            return (sidx_v if t % 2 == 0 else oidx_v).at[t // 2]

        ld = [issue(0, bufs[0], lsems[0]), issue(1, bufs[1], lsems[1])]
        for t in range(ntask):
            b = t % 2
            ld[b].wait()
            pltpu.sync_copy(bufs[b], shared.at[idx_row(t)], add=True)
            if t + 2 < ntask:
                ld[b] = issue(t + 2, bufs[b], lsems[b])
        plsc.subcore_barrier()

        pltpu.sync_copy(shared.at[pl.ds(s16 * WB, WB)],
                        pooled_out.at[pl.ds(s16 * WB, WB),
                                      pl.ds(c * 128, 128)])

    return sc_scatter


def _sc_scatter(ns, no, sidx, oidx):
    return _get_sc_scatter()(ns, no, sidx, oidx)

_PREC = jax.lax.Precision.DEFAULT


def _dot(a, b):
    return jax.lax.dot_general(a, b, (((1,), (0,)), ((), ())),
                               preferred_element_type=jnp.float32,
                               precision=_PREC)


def _embed_obj_body(objs_ref, boxes_ref, shapes_ref, eb_ref, es_ref,
                    wb_ref, bb_ref, ws_ref, bs_ref, ovb_ref, ovs_ref):
    idx = objs_ref[0, 0, :]
    oh = (idx[:, None] == lax.broadcasted_iota(jnp.int32, (1, 128), 1)
          ).astype(jnp.float32)
    evb = _dot(oh, eb_ref[...])
    evs = _dot(oh, es_ref[...])
    bv = _dot(boxes_ref[...], wb_ref[...]) + bb_ref[...]
    sv = _dot(shapes_ref[...], ws_ref[...]) + bs_ref[...]
    ovb_ref[...] = jnp.concatenate([evb, bv], axis=1)
    ovs_ref[...] = jnp.concatenate([evs, sv], axis=1)


def _embed_objs(objs, boxes_pad, shapes_gt, emb_b_pad, emb_s_pad,
                wb_pad, bb, ws, bs):
    nb = O // BO
    return pl.pallas_call(
        _embed_obj_body,
        grid=(nb,),
        in_specs=[
            pl.BlockSpec((1, 1, BO), lambda i: (i, 0, 0)),
            pl.BlockSpec((BO, 128), lambda i: (i, 0)),
            pl.BlockSpec((BO, 128), lambda i: (i, 0)),
            pl.BlockSpec((128, ED), lambda i: (0, 0)),
            pl.BlockSpec((128, ED), lambda i: (0, 0)),
            pl.BlockSpec((128, ED), lambda i: (0, 0)),
            pl.BlockSpec((1, ED), lambda i: (0, 0)),
            pl.BlockSpec((128, ED), lambda i: (0, 0)),
            pl.BlockSpec((1, ED), lambda i: (0, 0)),
        ],
        out_specs=[pl.BlockSpec((BO, DIN), lambda i: (i, 0)),
                   pl.BlockSpec((BO, DIN), lambda i: (i, 0))],
        out_shape=[jax.ShapeDtypeStruct((O, DIN), jnp.float32),
                   jax.ShapeDtypeStruct((O, DIN), jnp.float32)],
    )(objs.reshape(nb, 1, BO), boxes_pad, shapes_gt,
      emb_b_pad, emb_s_pad, wb_pad, bb, ws, bs)


def _embed_pred_body(p_ref, eb_ref, es_ref, pvb_ref, pvs_ref):
    idx = p_ref[0, 0, :]
    oh = (idx[:, None] == lax.broadcasted_iota(jnp.int32, (1, 128), 1)
          ).astype(jnp.float32)
    pvb_ref[...] = _dot(oh, eb_ref[...])
    pvs_ref[...] = _dot(oh, es_ref[...])


def _embed_preds(p, emb_b_pad, emb_s_pad):
    nb = TP // BT
    return pl.pallas_call(
        _embed_pred_body,
        grid=(nb,),
        in_specs=[
            pl.BlockSpec((1, 1, BT), lambda i: (i, 0, 0)),
            pl.BlockSpec((128, DIN), lambda i: (0, 0)),
            pl.BlockSpec((128, DIN), lambda i: (0, 0)),
        ],
        out_specs=[pl.BlockSpec((BT, DIN), lambda i: (i, 0)),
                   pl.BlockSpec((BT, DIN), lambda i: (i, 0))],
        out_shape=[jax.ShapeDtypeStruct((TP, DIN), jnp.float32),
                   jax.ShapeDtypeStruct((TP, DIN), jnp.float32)],
    )(p.reshape(nb, 1, BT), emb_b_pad, emb_s_pad)


def _stage_a_body(x_ref, ws_ref, wo_ref, a_ref, c_ref):
    x = x_ref[...]
    a_ref[...] = _dot(x, ws_ref[...])
    c_ref[...] = _dot(x, wo_ref[...])


def _stage_a(x, w1s, w1o):
    d = x.shape[1]
    nb = O // BO
    return pl.pallas_call(
        _stage_a_body,
        grid=(nb,),
        in_specs=[
            pl.BlockSpec((BO, d), lambda i: (i, 0)),
            pl.BlockSpec((d, H), lambda i: (0, 0)),
            pl.BlockSpec((d, H), lambda i: (0, 0)),
        ],
        out_specs=[pl.BlockSpec((BO, H), lambda i: (i, 0)),
                   pl.BlockSpec((BO, H), lambda i: (i, 0))],
        out_shape=[jax.ShapeDtypeStruct((O, H), jnp.float32),
                   jax.ShapeDtypeStruct((O, H), jnp.float32)],
    )(x, w1s, w1o)


def _stage_b_body(as_ref, cs_ref, pv_ref, w1p_ref, b1_ref, w2_ref, b2_ref,
                  ns_ref, np_ref, no_ref, *, dout):
    h = as_ref[...] + cs_ref[...] + _dot(pv_ref[...], w1p_ref[...]) + b1_ref[...]
    h = jnp.maximum(h, 0.0)
    nt = jnp.maximum(_dot(h, w2_ref[...]) + b2_ref[...], 0.0)
    ns_ref[...] = nt[:, :H]
    np_ref[...] = nt[:, H:H + dout]
    no_ref[...] = nt[:, H + dout:]


def _stage_b(As, Cs, pv, w1p, b1, w2, b2, dout):
    d = pv.shape[1]
    d2 = 2 * H + dout
    nb = TP // BT
    return pl.pallas_call(
        functools.partial(_stage_b_body, dout=dout),
        grid=(nb,),
        in_specs=[
            pl.BlockSpec((BT, H), lambda i: (i, 0)),
            pl.BlockSpec((BT, H), lambda i: (i, 0)),
            pl.BlockSpec((BT, d), lambda i: (i, 0)),
            pl.BlockSpec((d, H), lambda i: (0, 0)),
            pl.BlockSpec((1, H), lambda i: (0, 0)),
            pl.BlockSpec((H, d2), lambda i: (0, 0)),
            pl.BlockSpec((1, d2), lambda i: (0, 0)),
        ],
        out_specs=[pl.BlockSpec((BT, H), lambda i: (i, 0)),
                   pl.BlockSpec((BT, dout), lambda i: (i, 0)),
                   pl.BlockSpec((BT, H), lambda i: (i, 0))],
        out_shape=[jax.ShapeDtypeStruct((TP, H), jnp.float32),
                   jax.ShapeDtypeStruct((TP, dout), jnp.float32),
                   jax.ShapeDtypeStruct((TP, H), jnp.float32)],
    )(As, Cs, pv, w1p, b1, w2, b2)


def _stage_bd_body(os_ref, oo_ref, pv_ref, w1_ref, b1_ref, w2_ref, b2_ref,
                   ns_ref, np_ref, no_ref, *, dout):
    x = jnp.concatenate([os_ref[...], pv_ref[...], oo_ref[...]], axis=1)
    h = jnp.maximum(_dot(x, w1_ref[...]) + b1_ref[...], 0.0)
    nt = jnp.maximum(_dot(h, w2_ref[...]) + b2_ref[...], 0.0)
    ns_ref[...] = nt[:, :H]
    np_ref[...] = nt[:, H:H + dout]
    no_ref[...] = nt[:, H + dout:]


def _stage_bd(os_, oo, pv, w1, b1, w2, b2, dout):
    d = pv.shape[1]
    d2 = 2 * H + dout
    nb = TP // BT
    return pl.pallas_call(
        functools.partial(_stage_bd_body, dout=dout),
        grid=(nb,),
        in_specs=[
            pl.BlockSpec((BT, d), lambda i: (i, 0)),
            pl.BlockSpec((BT, d), lambda i: (i, 0)),
            pl.BlockSpec((BT, d), lambda i: (i, 0)),
            pl.BlockSpec((3 * d, H), lambda i: (0, 0)),
            pl.BlockSpec((1, H), lambda i: (0, 0)),
            pl.BlockSpec((H, d2), lambda i: (0, 0)),
            pl.BlockSpec((1, d2), lambda i: (0, 0)),
        ],
        out_specs=[pl.BlockSpec((BT, H), lambda i: (i, 0)),
                   pl.BlockSpec((BT, dout), lambda i: (i, 0)),
                   pl.BlockSpec((BT, H), lambda i: (i, 0))],
        out_shape=[jax.ShapeDtypeStruct((TP, H), jnp.float32),
                   jax.ShapeDtypeStruct((TP, dout), jnp.float32),
                   jax.ShapeDtypeStruct((TP, H), jnp.float32)],
    )(os_, oo, pv, w1, b1, w2, b2)


def _stage_c_body(pool_ref, cnt_ref, w3_ref, b3_ref, w4_ref, b4_ref, o_ref):
    x = pool_ref[...] / jnp.maximum(cnt_ref[...][:, :1], 1.0)
    h = jnp.maximum(_dot(x, w3_ref[...]) + b3_ref[...], 0.0)
    o_ref[...] = jnp.maximum(_dot(h, w4_ref[...]) + b4_ref[...], 0.0)


def _stage_c(pooled, counts_tab, w3, b3, w4, b4, dout):
    nb = O // BO
    return pl.pallas_call(
        _stage_c_body,
        grid=(nb,),
        in_specs=[
            pl.BlockSpec((BO, H), lambda i: (i, 0)),
            pl.BlockSpec((BO, H), lambda i: (i, 0)),
            pl.BlockSpec((H, H), lambda i: (0, 0)),
            pl.BlockSpec((1, H), lambda i: (0, 0)),
            pl.BlockSpec((H, dout), lambda i: (0, 0)),
            pl.BlockSpec((1, dout), lambda i: (0, 0)),
        ],
        out_specs=pl.BlockSpec((BO, dout), lambda i: (i, 0)),
        out_shape=jax.ShapeDtypeStruct((O, dout), jnp.float32),
    )(pooled, counts_tab, w3, b3, w4, b4)


def _stage_ca_body(pool_ref, cnt_ref, w3_ref, b3_ref, w4_ref, b4_ref,
                   ws_ref, wo_ref, o_ref, a_ref, c_ref):
    x = pool_ref[...] / jnp.maximum(cnt_ref[...][:, :1], 1.0)
    h = jnp.maximum(_dot(x, w3_ref[...]) + b3_ref[...], 0.0)
    y = jnp.maximum(_dot(h, w4_ref[...]) + b4_ref[...], 0.0)
    o_ref[...] = y
    a_ref[...] = _dot(y, ws_ref[...])
    c_ref[...] = _dot(y, wo_ref[...])


def _stage_ca(pooled, counts_tab, w3, b3, w4, b4, w1s, w1o, dout):
    nb = O // BO
    return pl.pallas_call(
        _stage_ca_body,
        grid=(nb,),
        in_specs=[
            pl.BlockSpec((BO, H), lambda i: (i, 0)),
            pl.BlockSpec((BO, H), lambda i: (i, 0)),
            pl.BlockSpec((H, H), lambda i: (0, 0)),
            pl.BlockSpec((1, H), lambda i: (0, 0)),
            pl.BlockSpec((H, dout), lambda i: (0, 0)),
            pl.BlockSpec((1, dout), lambda i: (0, 0)),
            pl.BlockSpec((dout, H), lambda i: (0, 0)),
            pl.BlockSpec((dout, H), lambda i: (0, 0)),
        ],
        out_specs=[pl.BlockSpec((BO, dout), lambda i: (i, 0)),
                   pl.BlockSpec((BO, H), lambda i: (i, 0)),
                   pl.BlockSpec((BO, H), lambda i: (i, 0))],
        out_shape=[jax.ShapeDtypeStruct((O, dout), jnp.float32),
                   jax.ShapeDtypeStruct((O, H), jnp.float32),
                   jax.ShapeDtypeStruct((O, H), jnp.float32)],
    )(pooled, counts_tab, w3, b3, w4, b4, w1s, w1o)


def _head_body(x_ref, w1_ref, b1_ref, w2_ref, b2_ref,
               wm_ref, bm_ref, wv_ref, bv_ref, mu_ref, lv_ref):
    h = jnp.maximum(_dot(x_ref[...], w1_ref[...]) + b1_ref[...], 0.0)
    hb = jnp.maximum(_dot(h, w2_ref[...]) + b2_ref[...], 0.0)
    mu_ref[...] = _dot(hb, wm_ref[...]) + bm_ref[...]
    lv_ref[...] = _dot(hb, wv_ref[...]) + bv_ref[...]


def _head(x, mv, mean, var):
    (w1, b1), (w2, b2) = mv
    (wm, bm), = mean
    (wv, bv), = var
    nb = O // BO
    return pl.pallas_call(
        _head_body,
        grid=(nb,),
        in_specs=[
            pl.BlockSpec((BO, DIN), lambda i: (i, 0)),
            pl.BlockSpec((DIN, H), lambda i: (0, 0)),
            pl.BlockSpec((1, H), lambda i: (0, 0)),
            pl.BlockSpec((H, DIN), lambda i: (0, 0)),
            pl.BlockSpec((1, DIN), lambda i: (0, 0)),
            pl.BlockSpec((DIN, ED), lambda i: (0, 0)),
            pl.BlockSpec((1, ED), lambda i: (0, 0)),
            pl.BlockSpec((DIN, ED), lambda i: (0, 0)),
            pl.BlockSpec((1, ED), lambda i: (0, 0)),
        ],
        out_specs=[pl.BlockSpec((BO, ED), lambda i: (i, 0)),
                   pl.BlockSpec((BO, ED), lambda i: (i, 0))],
        out_shape=[jax.ShapeDtypeStruct((O, ED), jnp.float32),
                   jax.ShapeDtypeStruct((O, ED), jnp.float32)],
    )(x, w1, b1.reshape(1, -1), w2, b2.reshape(1, -1),
      wm, bm.reshape(1, -1), wv, bv.reshape(1, -1))


def _layer_group(streams, idxs, counts_tab):
    sidx_g, oidx_g, sidx_sc, oidx_sc = idxs
    gathered = []
    for st in streams:
        if st["direct"]:
            gathered.append(_sc_gather(st["obj"], st["obj"], sidx_g, oidx_g))
        else:
            gathered.append(_sc_gather(st["ac"][0], st["ac"][1],
                                       sidx_g, oidx_g))
    edge_out = []
    for st, (ga, gc) in zip(streams, gathered):
        (w1, b1), (w2, b2) = st["gp"]["net1"]
        d = w1.shape[0] // 3
        if st["direct"]:
            edge_out.append(_stage_bd(ga, gc, st["pv"], w1,
                                      b1.reshape(1, -1), w2,
                                      b2.reshape(1, -1), st["dout"]))
        else:
            edge_out.append(_stage_b(ga, gc, st["pv"], w1[d:2 * d],
                                     b1.reshape(1, -1), w2,
                                     b2.reshape(1, -1), st["dout"]))
    pooled = [_sc_scatter(ns, no, sidx_sc, oidx_sc)
              for (ns, _, no) in edge_out]
    for st, (_, np_, __), pool in zip(streams, edge_out, pooled):
        (w3, b3), (w4, b4) = st["gp"]["net2"]
        st["pv"] = np_
        if st["direct"] or st["gp_next"] is None:
            st["obj"] = _stage_c(pool, counts_tab, w3, b3.reshape(1, -1),
                                 w4, b4.reshape(1, -1), st["dout"])
            st["ac"] = None
        else:
            w1n = st["gp_next"]["net1"][0][0]
            dn = w1n.shape[0] // 3
            st["obj"], a2, c2 = _stage_ca(
                pool, counts_tab, w3, b3.reshape(1, -1),
                w4, b4.reshape(1, -1), w1n[:dn], w1n[2 * dn:], st["dout"])
            st["ac"] = (a2, c2)
    return streams


def _prime(layers, obj_vecs):
    w1 = layers[0]["net1"][0][0]
    d = w1.shape[0] // 3
    return _stage_a(obj_vecs, w1[:d], w1[2 * d:])


def kernel(boxes_gt, shapes_gt, attributes, params, objs, triples):
    s = triples[:, 0]
    p = triples[:, 1]
    o = triples[:, 2]

    boxes_pad = jnp.pad(boxes_gt, ((0, 0), (0, 128 - boxes_gt.shape[1])))
    wb, bb = params["box_lin"][0]
    wb_pad = jnp.pad(wb, ((0, 128 - wb.shape[0]), (0, 0)))
    ws, bs = params["shape_lin"][0]
    eb_pad = jnp.pad(params["emb_obj_box"], ((0, 128 - NUM_OBJS - 1), (0, 0)))
    es_pad = jnp.pad(params["emb_obj_shape"], ((0, 128 - NUM_OBJS - 1), (0, 0)))
    peb_pad = jnp.pad(params["emb_pred_box"], ((0, 128 - NUM_PREDS), (0, 0)))
    pes_pad = jnp.pad(params["emb_pred_shape"], ((0, 128 - NUM_PREDS), (0, 0)))

    ovb, ovs = _embed_objs(objs, boxes_pad, shapes_gt, eb_pad, es_pad,
                           wb_pad, bb.reshape(1, -1), ws, bs.reshape(1, -1))
    p_pad = jnp.concatenate([p, jnp.zeros((TP - T,), p.dtype)])
    pvb, pvs = _embed_preds(p_pad, peb_pad, pes_pad)

    s32 = s.astype(jnp.int32)
    o32 = o.astype(jnp.int32)
    gpad = jnp.zeros((TP - T,), jnp.int32)
    spad = jnp.full((TP - T,), DUMP, jnp.int32)
    def _chunk_layout(idx, nblk, nch, nchp, cw):
        arr = jnp.zeros((nblk, nchp, cw), jnp.int32)
        arr = arr.at[:, :nch].set(idx.reshape(nblk, nch, cw))
        return arr.reshape(nblk * nchp, cw)

    sidx_g = _chunk_layout(jnp.concatenate([s32, gpad]), NW, GCH, GCHP, GCW)
    oidx_g = _chunk_layout(jnp.concatenate([o32, gpad]), NW, GCH, GCHP, GCW)
    sidx_sc = _chunk_layout(jnp.concatenate([s32, spad]), NS, SCH, SCHP, 128)
    oidx_sc = _chunk_layout(jnp.concatenate([o32, spad]), NS, SCH, SCHP, 128)
    idxs = (sidx_g, oidx_g, sidx_sc, oidx_sc)
    ones_tp = jnp.ones((TP, H), jnp.float32)
    counts_tab = _sc_scatter(ones_tp, ones_tp, sidx_sc, oidx_sc)

    lb = params["gconv_ec_box"]
    ls = params["gconv_ec_shape"]
    sb = {"direct": True, "ac": None, "pv": pvb, "dout": DIN, "obj": ovb}
    ss = {"direct": True, "ac": None, "pv": pvs, "dout": DIN, "obj": ovs}
    for i in range(len(lb)):
        sb["gp"], ss["gp"] = lb[i], ls[i]
        sb["gp_next"] = lb[i + 1] if i + 1 < len(lb) else None
        ss["gp_next"] = ls[i + 1] if i + 1 < len(ls) else None
        sb, ss = _layer_group([sb, ss], idxs, counts_tab)
    ovb, pvb = sb["obj"], sb["pv"]
    ovs, pvs = ss["obj"], ss["pv"]

    ov = jnp.concatenate([ovb, ovs], axis=1)
    pv = jnp.concatenate([pvb, pvs], axis=1)
    lsh = params["gconv_shared"]
    sh = {"direct": False, "ac": _prime(lsh, ov), "pv": pv, "dout": H,
          "obj": None}
    for i in range(len(lsh)):
        sh["gp"] = lsh[i]
        sh["gp_next"] = lsh[i + 1] if i + 1 < len(lsh) else None
        (sh,) = _layer_group([sh], idxs, counts_tab)
    ov = sh["obj"]

    mu_box, lv_box = _head(ov[:, :DIN], params["box_mean_var"],
                           params["box_mean"], params["box_var"])
    mu_shape, lv_shape = _head(ov[:, DIN:], params["shape_mean_var"],
                               params["shape_mean"], params["shape_var"])
    return (mu_box, lv_box, mu_shape, lv_shape)


# speedup vs baseline: 1.0021x; 1.0021x over previous
import functools

import jax
import jax.numpy as jnp
from jax import lax
from jax.experimental import pallas as pl
from jax.experimental.pallas import tpu as pltpu
from jax.experimental.pallas import tpu_sc as plsc

ED = 64
NUM_OBJS = 40
NUM_PREDS = 26
O = 10000
T = 20000
DIN = 2 * ED
H = 4 * ED

BO = 1000
BT = 1024

NC, NS = 2, 16
NW = NC * NS
TP = 20480
OP = 10240
DUMP = O
GCW = 64
GNB = 6
GCH = TP // NW // GCW
GCHP = 16
SCH = TP // NS // 128
SCHP = 16
WB = OP // NS

@functools.lru_cache(maxsize=None)
def _get_sc_gather(width):
    mesh = plsc.VectorSubcoreMesh(core_axis_name="c", subcore_axis_name="s")

    @functools.partial(
        pl.kernel, mesh=mesh,
        out_type=[jax.ShapeDtypeStruct((TP, width), jnp.float32),
                  jax.ShapeDtypeStruct((TP, width), jnp.float32)],
        scratch_types=(
            [pltpu.VMEM((GCHP, GCW), jnp.int32),
             pltpu.VMEM((GCHP, GCW), jnp.int32)]
            + [pltpu.VMEM((GCW, width), jnp.float32)] * GNB
            + [pltpu.SemaphoreType.DMA] * (2 * GNB)),
    )
    def sc_gather(a_hbm, c_hbm, sidx_hbm, oidx_hbm, as_out, cs_out,
                  sidx_v, oidx_v, *bufs_sems):
        wid = lax.axis_index("s") * NC + lax.axis_index("c")
        pltpu.sync_copy(sidx_hbm.at[pl.ds(wid * GCHP, GCHP)], sidx_v)
        pltpu.sync_copy(oidx_hbm.at[pl.ds(wid * GCHP, GCHP)], oidx_v)
        bufs = bufs_sems[:GNB]
        gsems = bufs_sems[GNB:2 * GNB]
        wsems = bufs_sems[2 * GNB:]
        nbuf = GNB
        ntask = 2 * GCH

        def issue(t, buf, sem):
            j = t // 2
            if t % 2 == 0:
                return pltpu.async_copy(a_hbm.at[sidx_v.at[j]], buf, sem)
            return pltpu.async_copy(c_hbm.at[oidx_v.at[j]], buf, sem)

        def out_dst(t):
            base = wid * (GCH * GCW) + (t // 2) * GCW
            dst = as_out if t % 2 == 0 else cs_out
            return dst.at[pl.ds(base, GCW)]

        gd = [issue(b, bufs[b], gsems[b]) for b in range(nbuf)]
        wd = [None] * nbuf
        for t in range(ntask):
            b = t % nbuf
            gd[b].wait()
            wd[b] = pltpu.async_copy(bufs[b], out_dst(t), wsems[b])
            if t + nbuf < ntask:
                wd[b].wait()
                gd[b] = issue(t + nbuf, bufs[b], gsems[b])
        for b in range(nbuf):
            if wd[b] is not None:
                wd[b].wait()

    return sc_gather


def _sc_gather(a, c, sidx, oidx):
    return _get_sc_gather(a.shape[1])(a, c, sidx, oidx)


@functools.lru_cache(maxsize=None)
def _get_sc_scatter():
    mesh = plsc.VectorSubcoreMesh(core_axis_name="c", subcore_axis_name="s")

    @functools.partial(
        pl.kernel, mesh=mesh,
        out_type=jax.ShapeDtypeStruct((OP, H), jnp.float32),
        scratch_types=[pltpu.VMEM((SCHP, 128), jnp.int32),
                       pltpu.VMEM((SCHP, 128), jnp.int32),
                       pltpu.VMEM((128, 128), jnp.float32),
                       pltpu.VMEM((128, 128), jnp.float32),
                       pltpu.SemaphoreType.DMA,
                       pltpu.SemaphoreType.DMA,
                       pltpu.VMEM_SHARED((OP, 128), jnp.float32)],
    )
    def sc_scatter(ns_hbm, no_hbm, sidx_hbm, oidx_hbm, pooled_out,
                   sidx_v, oidx_v, buf0_v, buf1_v, l0, l1, shared):
        c = lax.axis_index("c")
        s16 = lax.axis_index("s")
        rows0 = s16 * (SCH * 128)
        pltpu.sync_copy(sidx_hbm.at[pl.ds(s16 * SCHP, SCHP)], sidx_v)
        pltpu.sync_copy(oidx_hbm.at[pl.ds(s16 * SCHP, SCHP)], oidx_v)

        zv = jnp.zeros((16,), jnp.float32)

        def zfill(r, carry):
            for k in range(8):
                buf0_v[r, pl.ds(k * 16, 16)] = zv
            return carry

        lax.fori_loop(0, 128, zfill, 0)
        zds = [pltpu.async_copy(buf0_v,
                                shared.at[pl.ds(s16 * WB + w * 128, 128)],
                                l0)
               for w in range(WB // 128)]
        for zd in zds:
            zd.wait()
        plsc.subcore_barrier()

        bufs = (buf0_v, buf1_v)
        lsems = (l0, l1)
        ntask = 2 * SCH

        def issue(t, buf, sem):
            r = rows0 + (t // 2) * 128
            src = ns_hbm if t % 2 == 0 else no_hbm
            return pltpu.async_copy(
                src.at[pl.ds(r, 128), pl.ds(c * 128, 128)], buf, sem)

        def idx_row(t):
            return (sidx_v if t % 2 == 0 else oidx_v).at[t // 2]

        ld = [issue(0, bufs[0], lsems[0]), issue(1, bufs[1], lsems[1])]
        for t in range(ntask):
            b = t % 2
            ld[b].wait()
            pltpu.sync_copy(bufs[b], shared.at[idx_row(t)], add=True)
            if t + 2 < ntask:
                ld[b] = issue(t + 2, bufs[b], lsems[b])
        plsc.subcore_barrier()

        pltpu.sync_copy(shared.at[pl.ds(s16 * WB, WB)],
                        pooled_out.at[pl.ds(s16 * WB, WB),
                                      pl.ds(c * 128, 128)])

    return sc_scatter


def _sc_scatter(ns, no, sidx, oidx):
    return _get_sc_scatter()(ns, no, sidx, oidx)

_PREC = jax.lax.Precision.DEFAULT


def _dot(a, b):
    return jax.lax.dot_general(a, b, (((1,), (0,)), ((), ())),
                               preferred_element_type=jnp.float32,
                               precision=_PREC)


def _embed_obj_body(objs_ref, boxes_ref, shapes_ref, eb_ref, es_ref,
                    wb_ref, bb_ref, ws_ref, bs_ref, ovb_ref, ovs_ref):
    idx = objs_ref[0, 0, :]
    oh = (idx[:, None] == lax.broadcasted_iota(jnp.int32, (1, 128), 1)
          ).astype(jnp.float32)
    evb = _dot(oh, eb_ref[...])
    evs = _dot(oh, es_ref[...])
    bv = _dot(boxes_ref[...], wb_ref[...]) + bb_ref[...]
    sv = _dot(shapes_ref[...], ws_ref[...]) + bs_ref[...]
    ovb_ref[...] = jnp.concatenate([evb, bv], axis=1)
    ovs_ref[...] = jnp.concatenate([evs, sv], axis=1)


def _embed_objs(objs, boxes_pad, shapes_gt, emb_b_pad, emb_s_pad,
                wb_pad, bb, ws, bs):
    nb = O // BO
    return pl.pallas_call(
        _embed_obj_body,
        grid=(nb,),
        in_specs=[
            pl.BlockSpec((1, 1, BO), lambda i: (i, 0, 0)),
            pl.BlockSpec((BO, 128), lambda i: (i, 0)),
            pl.BlockSpec((BO, 128), lambda i: (i, 0)),
            pl.BlockSpec((128, ED), lambda i: (0, 0)),
            pl.BlockSpec((128, ED), lambda i: (0, 0)),
            pl.BlockSpec((128, ED), lambda i: (0, 0)),
            pl.BlockSpec((1, ED), lambda i: (0, 0)),
            pl.BlockSpec((128, ED), lambda i: (0, 0)),
            pl.BlockSpec((1, ED), lambda i: (0, 0)),
        ],
        out_specs=[pl.BlockSpec((BO, DIN), lambda i: (i, 0)),
                   pl.BlockSpec((BO, DIN), lambda i: (i, 0))],
        out_shape=[jax.ShapeDtypeStruct((O, DIN), jnp.float32),
                   jax.ShapeDtypeStruct((O, DIN), jnp.float32)],
    )(objs.reshape(nb, 1, BO), boxes_pad, shapes_gt,
      emb_b_pad, emb_s_pad, wb_pad, bb, ws, bs)


def _embed_pred_body(p_ref, eb_ref, es_ref, pvb_ref, pvs_ref):
    idx = p_ref[0, 0, :]
    oh = (idx[:, None] == lax.broadcasted_iota(jnp.int32, (1, 128), 1)
          ).astype(jnp.float32)
    pvb_ref[...] = _dot(oh, eb_ref[...])
    pvs_ref[...] = _dot(oh, es_ref[...])


def _embed_preds(p, emb_b_pad, emb_s_pad):
    nb = TP // BT
    return pl.pallas_call(
        _embed_pred_body,
        grid=(nb,),
        in_specs=[
            pl.BlockSpec((1, 1, BT), lambda i: (i, 0, 0)),
            pl.BlockSpec((128, DIN), lambda i: (0, 0)),
            pl.BlockSpec((128, DIN), lambda i: (0, 0)),
        ],
        out_specs=[pl.BlockSpec((BT, DIN), lambda i: (i, 0)),
                   pl.BlockSpec((BT, DIN), lambda i: (i, 0))],
        out_shape=[jax.ShapeDtypeStruct((TP, DIN), jnp.float32),
                   jax.ShapeDtypeStruct((TP, DIN), jnp.float32)],
    )(p.reshape(nb, 1, BT), emb_b_pad, emb_s_pad)


def _stage_a_body(x_ref, ws_ref, wo_ref, a_ref, c_ref):
    x = x_ref[...]
    a_ref[...] = _dot(x, ws_ref[...])
    c_ref[...] = _dot(x, wo_ref[...])


def _stage_a(x, w1s, w1o):
    d = x.shape[1]
    nb = O // BO
    return pl.pallas_call(
        _stage_a_body,
        grid=(nb,),
        in_specs=[
            pl.BlockSpec((BO, d), lambda i: (i, 0)),
            pl.BlockSpec((d, H), lambda i: (0, 0)),
            pl.BlockSpec((d, H), lambda i: (0, 0)),
        ],
        out_specs=[pl.BlockSpec((BO, H), lambda i: (i, 0)),
                   pl.BlockSpec((BO, H), lambda i: (i, 0))],
        out_shape=[jax.ShapeDtypeStruct((O, H), jnp.float32),
                   jax.ShapeDtypeStruct((O, H), jnp.float32)],
    )(x, w1s, w1o)


def _stage_b_body(as_ref, cs_ref, pv_ref, w1p_ref, b1_ref, w2_ref, b2_ref,
                  ns_ref, np_ref, no_ref, *, dout):
    h = as_ref[...] + cs_ref[...] + _dot(pv_ref[...], w1p_ref[...]) + b1_ref[...]
    h = jnp.maximum(h, 0.0)
    nt = jnp.maximum(_dot(h, w2_ref[...]) + b2_ref[...], 0.0)
    ns_ref[...] = nt[:, :H]
    np_ref[...] = nt[:, H:H + dout]
    no_ref[...] = nt[:, H + dout:]


def _stage_b(As, Cs, pv, w1p, b1, w2, b2, dout):
    d = pv.shape[1]
    d2 = 2 * H + dout
    nb = TP // BT
    return pl.pallas_call(
        functools.partial(_stage_b_body, dout=dout),
        grid=(nb,),
        in_specs=[
            pl.BlockSpec((BT, H), lambda i: (i, 0)),
            pl.BlockSpec((BT, H), lambda i: (i, 0)),
            pl.BlockSpec((BT, d), lambda i: (i, 0)),
            pl.BlockSpec((d, H), lambda i: (0, 0)),
            pl.BlockSpec((1, H), lambda i: (0, 0)),
            pl.BlockSpec((H, d2), lambda i: (0, 0)),
            pl.BlockSpec((1, d2), lambda i: (0, 0)),
        ],
        out_specs=[pl.BlockSpec((BT, H), lambda i: (i, 0)),
                   pl.BlockSpec((BT, dout), lambda i: (i, 0)),
                   pl.BlockSpec((BT, H), lambda i: (i, 0))],
        out_shape=[jax.ShapeDtypeStruct((TP, H), jnp.float32),
                   jax.ShapeDtypeStruct((TP, dout), jnp.float32),
                   jax.ShapeDtypeStruct((TP, H), jnp.float32)],
    )(As, Cs, pv, w1p, b1, w2, b2)


def _stage_bd_body(os_ref, oo_ref, pv_ref, w1_ref, b1_ref, w2_ref, b2_ref,
                   ns_ref, np_ref, no_ref, *, dout):
    x = jnp.concatenate([os_ref[...], pv_ref[...], oo_ref[...]], axis=1)
    h = jnp.maximum(_dot(x, w1_ref[...]) + b1_ref[...], 0.0)
    nt = jnp.maximum(_dot(h, w2_ref[...]) + b2_ref[...], 0.0)
    ns_ref[...] = nt[:, :H]
    np_ref[...] = nt[:, H:H + dout]
    no_ref[...] = nt[:, H + dout:]


def _stage_bd(os_, oo, pv, w1, b1, w2, b2, dout):
    d = pv.shape[1]
    d2 = 2 * H + dout
    nb = TP // BT
    return pl.pallas_call(
        functools.partial(_stage_bd_body, dout=dout),
        grid=(nb,),
        in_specs=[
            pl.BlockSpec((BT, d), lambda i: (i, 0)),
            pl.BlockSpec((BT, d), lambda i: (i, 0)),
            pl.BlockSpec((BT, d), lambda i: (i, 0)),
            pl.BlockSpec((3 * d, H), lambda i: (0, 0)),
            pl.BlockSpec((1, H), lambda i: (0, 0)),
            pl.BlockSpec((H, d2), lambda i: (0, 0)),
            pl.BlockSpec((1, d2), lambda i: (0, 0)),
        ],
        out_specs=[pl.BlockSpec((BT, H), lambda i: (i, 0)),
                   pl.BlockSpec((BT, dout), lambda i: (i, 0)),
                   pl.BlockSpec((BT, H), lambda i: (i, 0))],
        out_shape=[jax.ShapeDtypeStruct((TP, H), jnp.float32),
                   jax.ShapeDtypeStruct((TP, dout), jnp.float32),
                   jax.ShapeDtypeStruct((TP, H), jnp.float32)],
    )(os_, oo, pv, w1, b1, w2, b2)


def _stage_c_body(pool_ref, cnt_ref, w3_ref, b3_ref, w4_ref, b4_ref, o_ref):
    x = pool_ref[...] / jnp.maximum(cnt_ref[...][:, :1], 1.0)
    h = jnp.maximum(_dot(x, w3_ref[...]) + b3_ref[...], 0.0)
    o_ref[...] = jnp.maximum(_dot(h, w4_ref[...]) + b4_ref[...], 0.0)


def _stage_c(pooled, counts_tab, w3, b3, w4, b4, dout):
    nb = O // BO
    return pl.pallas_call(
        _stage_c_body,
        grid=(nb,),
        in_specs=[
            pl.BlockSpec((BO, H), lambda i: (i, 0)),
            pl.BlockSpec((BO, H), lambda i: (i, 0)),
            pl.BlockSpec((H, H), lambda i: (0, 0)),
            pl.BlockSpec((1, H), lambda i: (0, 0)),
            pl.BlockSpec((H, dout), lambda i: (0, 0)),
            pl.BlockSpec((1, dout), lambda i: (0, 0)),
        ],
        out_specs=pl.BlockSpec((BO, dout), lambda i: (i, 0)),
        out_shape=jax.ShapeDtypeStruct((O, dout), jnp.float32),
    )(pooled, counts_tab, w3, b3, w4, b4)


def _stage_ca_body(pool_ref, cnt_ref, w3_ref, b3_ref, w4_ref, b4_ref,
                   ws_ref, wo_ref, o_ref, a_ref, c_ref):
    x = pool_ref[...] / jnp.maximum(cnt_ref[...][:, :1], 1.0)
    h = jnp.maximum(_dot(x, w3_ref[...]) + b3_ref[...], 0.0)
    y = jnp.maximum(_dot(h, w4_ref[...]) + b4_ref[...], 0.0)
    o_ref[...] = y
    a_ref[...] = _dot(y, ws_ref[...])
    c_ref[...] = _dot(y, wo_ref[...])


def _stage_ca(pooled, counts_tab, w3, b3, w4, b4, w1s, w1o, dout):
    nb = O // BO
    return pl.pallas_call(
        _stage_ca_body,
        grid=(nb,),
        in_specs=[
            pl.BlockSpec((BO, H), lambda i: (i, 0)),
            pl.BlockSpec((BO, H), lambda i: (i, 0)),
            pl.BlockSpec((H, H), lambda i: (0, 0)),
            pl.BlockSpec((1, H), lambda i: (0, 0)),
            pl.BlockSpec((H, dout), lambda i: (0, 0)),
            pl.BlockSpec((1, dout), lambda i: (0, 0)),
            pl.BlockSpec((dout, H), lambda i: (0, 0)),
            pl.BlockSpec((dout, H), lambda i: (0, 0)),
        ],
        out_specs=[pl.BlockSpec((BO, dout), lambda i: (i, 0)),
                   pl.BlockSpec((BO, H), lambda i: (i, 0)),
                   pl.BlockSpec((BO, H), lambda i: (i, 0))],
        out_shape=[jax.ShapeDtypeStruct((O, dout), jnp.float32),
                   jax.ShapeDtypeStruct((O, H), jnp.float32),
                   jax.ShapeDtypeStruct((O, H), jnp.float32)],
    )(pooled, counts_tab, w3, b3, w4, b4, w1s, w1o)


def _head_body(x_ref, w1_ref, b1_ref, w2_ref, b2_ref,
               wm_ref, bm_ref, wv_ref, bv_ref, mu_ref, lv_ref):
    h = jnp.maximum(_dot(x_ref[...], w1_ref[...]) + b1_ref[...], 0.0)
    hb = jnp.maximum(_dot(h, w2_ref[...]) + b2_ref[...], 0.0)
    mu_ref[...] = _dot(hb, wm_ref[...]) + bm_ref[...]
    lv_ref[...] = _dot(hb, wv_ref[...]) + bv_ref[...]


def _head(x, mv, mean, var):
    (w1, b1), (w2, b2) = mv
    (wm, bm), = mean
    (wv, bv), = var
    nb = O // BO
    return pl.pallas_call(
        _head_body,
        grid=(nb,),
        in_specs=[
            pl.BlockSpec((BO, DIN), lambda i: (i, 0)),
            pl.BlockSpec((DIN, H), lambda i: (0, 0)),
            pl.BlockSpec((1, H), lambda i: (0, 0)),
            pl.BlockSpec((H, DIN), lambda i: (0, 0)),
            pl.BlockSpec((1, DIN), lambda i: (0, 0)),
            pl.BlockSpec((DIN, ED), lambda i: (0, 0)),
            pl.BlockSpec((1, ED), lambda i: (0, 0)),
            pl.BlockSpec((DIN, ED), lambda i: (0, 0)),
            pl.BlockSpec((1, ED), lambda i: (0, 0)),
        ],
        out_specs=[pl.BlockSpec((BO, ED), lambda i: (i, 0)),
                   pl.BlockSpec((BO, ED), lambda i: (i, 0))],
        out_shape=[jax.ShapeDtypeStruct((O, ED), jnp.float32),
                   jax.ShapeDtypeStruct((O, ED), jnp.float32)],
    )(x, w1, b1.reshape(1, -1), w2, b2.reshape(1, -1),
      wm, bm.reshape(1, -1), wv, bv.reshape(1, -1))


def _layer_group(streams, idxs, counts_tab):
    sidx_g, oidx_g, sidx_sc, oidx_sc = idxs
    gathered = []
    for st in streams:
        if st["direct"]:
            gathered.append(_sc_gather(st["obj"], st["obj"], sidx_g, oidx_g))
        else:
            gathered.append(_sc_gather(st["ac"][0], st["ac"][1],
                                       sidx_g, oidx_g))
    edge_out = []
    for st, (ga, gc) in zip(streams, gathered):
        (w1, b1), (w2, b2) = st["gp"]["net1"]
        d = w1.shape[0] // 3
        if st["direct"]:
            edge_out.append(_stage_bd(ga, gc, st["pv"], w1,
                                      b1.reshape(1, -1), w2,
                                      b2.reshape(1, -1), st["dout"]))
        else:
            edge_out.append(_stage_b(ga, gc, st["pv"], w1[d:2 * d],
                                     b1.reshape(1, -1), w2,
                                     b2.reshape(1, -1), st["dout"]))
    pooled = [_sc_scatter(ns, no, sidx_sc, oidx_sc)
              for (ns, _, no) in edge_out]
    for st, (_, np_, __), pool in zip(streams, edge_out, pooled):
        (w3, b3), (w4, b4) = st["gp"]["net2"]
        st["pv"] = np_
        if st["direct"] or st["gp_next"] is None:
            st["obj"] = _stage_c(pool, counts_tab, w3, b3.reshape(1, -1),
                                 w4, b4.reshape(1, -1), st["dout"])
            st["ac"] = None
        else:
            w1n = st["gp_next"]["net1"][0][0]
            dn = w1n.shape[0] // 3
            st["obj"], a2, c2 = _stage_ca(
                pool, counts_tab, w3, b3.reshape(1, -1),
                w4, b4.reshape(1, -1), w1n[:dn], w1n[2 * dn:], st["dout"])
            st["ac"] = (a2, c2)
    return streams


def _prime(layers, obj_vecs):
    w1 = layers[0]["net1"][0][0]
    d = w1.shape[0] // 3
    return _stage_a(obj_vecs, w1[:d], w1[2 * d:])


def kernel(boxes_gt, shapes_gt, attributes, params, objs, triples):
    s = triples[:, 0]
    p = triples[:, 1]
    o = triples[:, 2]

    boxes_pad = jnp.pad(boxes_gt, ((0, 0), (0, 128 - boxes_gt.shape[1])))
    wb, bb = params["box_lin"][0]
    wb_pad = jnp.pad(wb, ((0, 128 - wb.shape[0]), (0, 0)))
    ws, bs = params["shape_lin"][0]
    eb_pad = jnp.pad(params["emb_obj_box"], ((0, 128 - NUM_OBJS - 1), (0, 0)))
    es_pad = jnp.pad(params["emb_obj_shape"], ((0, 128 - NUM_OBJS - 1), (0, 0)))
    peb_pad = jnp.pad(params["emb_pred_box"], ((0, 128 - NUM_PREDS), (0, 0)))
    pes_pad = jnp.pad(params["emb_pred_shape"], ((0, 128 - NUM_PREDS), (0, 0)))

    ovb, ovs = _embed_objs(objs, boxes_pad, shapes_gt, eb_pad, es_pad,
                           wb_pad, bb.reshape(1, -1), ws, bs.reshape(1, -1))
    p_pad = jnp.concatenate([p, jnp.zeros((TP - T,), p.dtype)])
    pvb, pvs = _embed_preds(p_pad, peb_pad, pes_pad)

    s32 = s.astype(jnp.int32)
    o32 = o.astype(jnp.int32)
    gpad = jnp.zeros((TP - T,), jnp.int32)
    spad = jnp.full((TP - T,), DUMP, jnp.int32)
    def _chunk_layout(idx, nblk, nch, nchp, cw):
        arr = jnp.zeros((nblk, nchp, cw), jnp.int32)
        arr = arr.at[:, :nch].set(idx.reshape(nblk, nch, cw))
        return arr.reshape(nblk * nchp, cw)

    sidx_g = _chunk_layout(jnp.concatenate([s32, gpad]), NW, GCH, GCHP, GCW)
    oidx_g = _chunk_layout(jnp.concatenate([o32, gpad]), NW, GCH, GCHP, GCW)
    sidx_sc = _chunk_layout(jnp.concatenate([s32, spad]), NS, SCH, SCHP, 128)
    oidx_sc = _chunk_layout(jnp.concatenate([o32, spad]), NS, SCH, SCHP, 128)
    idxs = (sidx_g, oidx_g, sidx_sc, oidx_sc)
    ones_tp = jnp.ones((TP, H), jnp.float32)
    counts_tab = _sc_scatter(ones_tp, ones_tp, sidx_sc, oidx_sc)

    lb = params["gconv_ec_box"]
    ls = params["gconv_ec_shape"]
    sb = {"direct": True, "ac": None, "pv": pvb, "dout": DIN, "obj": ovb}
    ss = {"direct": True, "ac": None, "pv": pvs, "dout": DIN, "obj": ovs}
    for i in range(len(lb)):
        sb["gp"], ss["gp"] = lb[i], ls[i]
        sb["gp_next"] = lb[i + 1] if i + 1 < len(lb) else None
        ss["gp_next"] = ls[i + 1] if i + 1 < len(ls) else None
        sb, ss = _layer_group([sb, ss], idxs, counts_tab)
    ovb, pvb = sb["obj"], sb["pv"]
    ovs, pvs = ss["obj"], ss["pv"]

    ov = jnp.concatenate([ovb, ovs], axis=1)
    pv = jnp.concatenate([pvb, pvs], axis=1)
    lsh = params["gconv_shared"]
    sh = {"direct": False, "ac": _prime(lsh, ov), "pv": pv, "dout": H,
          "obj": None}
    for i in range(len(lsh)):
        sh["gp"] = lsh[i]
        sh["gp_next"] = lsh[i + 1] if i + 1 < len(lsh) else None
        (sh,) = _layer_group([sh], idxs, counts_tab)
    ov = sh["obj"]

    mu_box, lv_box = _head(ov[:, :DIN], params["box_mean_var"],
                           params["box_mean"], params["box_var"])
    mu_shape, lv_shape = _head(ov[:, DIN:], params["shape_mean_var"],
                               params["shape_mean"], params["shape_var"])
    return (mu_box, lv_box, mu_shape, lv_shape)


# speedup vs baseline: 1.0073x; 1.0052x over previous
import functools

import jax
import jax.numpy as jnp
from jax import lax
from jax.experimental import pallas as pl
from jax.experimental.pallas import tpu as pltpu
from jax.experimental.pallas import tpu_sc as plsc

ED = 64
NUM_OBJS = 40
NUM_PREDS = 26
O = 10000
T = 20000
DIN = 2 * ED
H = 4 * ED

BO = 1000
BT = 1024

NC, NS = 2, 16
NW = NC * NS
TP = 20480
OP = 10240
DUMP = O
GCW = 64
GNB = 6
GCH = TP // NW // GCW
GCHP = 16
SCH = TP // NS // 128
SCHP = 16
WB = OP // NS

@functools.lru_cache(maxsize=None)
def _get_sc_gather(width):
    mesh = plsc.VectorSubcoreMesh(core_axis_name="c", subcore_axis_name="s")

    @functools.partial(
        pl.kernel, mesh=mesh,
        out_type=[jax.ShapeDtypeStruct((TP, width), jnp.float32),
                  jax.ShapeDtypeStruct((TP, width), jnp.float32)],
        scratch_types=(
            [pltpu.VMEM((GCHP, GCW), jnp.int32),
             pltpu.VMEM((GCHP, GCW), jnp.int32)]
            + [pltpu.VMEM((GCW, width), jnp.float32)] * GNB
            + [pltpu.SemaphoreType.DMA] * (2 * GNB)),
    )
    def sc_gather(a_hbm, c_hbm, sidx_hbm, oidx_hbm, as_out, cs_out,
                  sidx_v, oidx_v, *bufs_sems):
        wid = lax.axis_index("s") * NC + lax.axis_index("c")
        pltpu.sync_copy(sidx_hbm.at[pl.ds(wid * GCHP, GCHP)], sidx_v)
        pltpu.sync_copy(oidx_hbm.at[pl.ds(wid * GCHP, GCHP)], oidx_v)
        bufs = bufs_sems[:GNB]
        gsems = bufs_sems[GNB:2 * GNB]
        wsems = bufs_sems[2 * GNB:]
        nbuf = GNB
        ntask = 2 * GCH

        def issue(t, buf, sem):
            j = t // 2
            if t % 2 == 0:
                return pltpu.async_copy(a_hbm.at[sidx_v.at[j]], buf, sem)
            return pltpu.async_copy(c_hbm.at[oidx_v.at[j]], buf, sem)

        def out_dst(t):
            base = wid * (GCH * GCW) + (t // 2) * GCW
            dst = as_out if t % 2 == 0 else cs_out
            return dst.at[pl.ds(base, GCW)]

        gd = [issue(b, bufs[b], gsems[b]) for b in range(nbuf)]
        wd = [None] * nbuf
        for t in range(ntask):
            b = t % nbuf
            gd[b].wait()
            wd[b] = pltpu.async_copy(bufs[b], out_dst(t), wsems[b])
            if t + nbuf < ntask:
                wd[b].wait()
                gd[b] = issue(t + nbuf, bufs[b], gsems[b])
        for b in range(nbuf):
            if wd[b] is not None:
                wd[b].wait()

    return sc_gather


def _sc_gather(a, c, sidx, oidx):
    return _get_sc_gather(a.shape[1])(a, c, sidx, oidx)


@functools.lru_cache(maxsize=None)
def _get_sc_scatter():
    mesh = plsc.VectorSubcoreMesh(core_axis_name="c", subcore_axis_name="s")

    @functools.partial(
        pl.kernel, mesh=mesh,
        out_type=jax.ShapeDtypeStruct((OP, H), jnp.float32),
        scratch_types=[pltpu.VMEM((SCHP, 128), jnp.int32),
                       pltpu.VMEM((SCHP, 128), jnp.int32),
                       pltpu.VMEM((128, 128), jnp.float32),
                       pltpu.VMEM((128, 128), jnp.float32),
                       pltpu.SemaphoreType.DMA,
                       pltpu.SemaphoreType.DMA,
                       pltpu.VMEM_SHARED((OP, 128), jnp.float32)],
    )
    def sc_scatter(ns_hbm, no_hbm, sidx_hbm, oidx_hbm, zeros_hbm, pooled_out,
                   sidx_v, oidx_v, buf0_v, buf1_v, l0, l1, shared):
        c = lax.axis_index("c")
        s16 = lax.axis_index("s")
        rows0 = s16 * (SCH * 128)
        pltpu.sync_copy(sidx_hbm.at[pl.ds(s16 * SCHP, SCHP)], sidx_v)
        pltpu.sync_copy(oidx_hbm.at[pl.ds(s16 * SCHP, SCHP)], oidx_v)

        pltpu.sync_copy(zeros_hbm.at[pl.ds(s16 * WB, WB)],
                        shared.at[pl.ds(s16 * WB, WB)])
        plsc.subcore_barrier()

        bufs = (buf0_v, buf1_v)
        lsems = (l0, l1)
        ntask = 2 * SCH

        def issue(t, buf, sem):
            r = rows0 + (t // 2) * 128
            src = ns_hbm if t % 2 == 0 else no_hbm
            return pltpu.async_copy(
                src.at[pl.ds(r, 128), pl.ds(c * 128, 128)], buf, sem)

        def idx_row(t):
            return (sidx_v if t % 2 == 0 else oidx_v).at[t // 2]

        ld = [issue(0, bufs[0], lsems[0]), issue(1, bufs[1], lsems[1])]
        for t in range(ntask):
            b = t % 2
            ld[b].wait()
            pltpu.sync_copy(bufs[b], shared.at[idx_row(t)], add=True)
            if t + 2 < ntask:
                ld[b] = issue(t + 2, bufs[b], lsems[b])
        plsc.subcore_barrier()

        pltpu.sync_copy(shared.at[pl.ds(s16 * WB, WB)],
                        pooled_out.at[pl.ds(s16 * WB, WB),
                                      pl.ds(c * 128, 128)])

    return sc_scatter


def _sc_scatter(ns, no, sidx, oidx, zeros_hbm):
    return _get_sc_scatter()(ns, no, sidx, oidx, zeros_hbm)

_PREC = jax.lax.Precision.DEFAULT


def _dot(a, b):
    return jax.lax.dot_general(a, b, (((1,), (0,)), ((), ())),
                               preferred_element_type=jnp.float32,
                               precision=_PREC)


def _embed_obj_body(objs_ref, boxes_ref, shapes_ref, eb_ref, es_ref,
                    wb_ref, bb_ref, ws_ref, bs_ref, ovb_ref, ovs_ref):
    idx = objs_ref[0, 0, :]
    oh = (idx[:, None] == lax.broadcasted_iota(jnp.int32, (1, 128), 1)
          ).astype(jnp.float32)
    evb = _dot(oh, eb_ref[...])
    evs = _dot(oh, es_ref[...])
    bv = _dot(boxes_ref[...], wb_ref[...]) + bb_ref[...]
    sv = _dot(shapes_ref[...], ws_ref[...]) + bs_ref[...]
    ovb_ref[...] = jnp.concatenate([evb, bv], axis=1)
    ovs_ref[...] = jnp.concatenate([evs, sv], axis=1)


def _embed_objs(objs, boxes_pad, shapes_gt, emb_b_pad, emb_s_pad,
                wb_pad, bb, ws, bs):
    nb = O // BO
    return pl.pallas_call(
        _embed_obj_body,
        grid=(nb,),
        in_specs=[
            pl.BlockSpec((1, 1, BO), lambda i: (i, 0, 0)),
            pl.BlockSpec((BO, 128), lambda i: (i, 0)),
            pl.BlockSpec((BO, 128), lambda i: (i, 0)),
            pl.BlockSpec((128, ED), lambda i: (0, 0)),
            pl.BlockSpec((128, ED), lambda i: (0, 0)),
            pl.BlockSpec((128, ED), lambda i: (0, 0)),
            pl.BlockSpec((1, ED), lambda i: (0, 0)),
            pl.BlockSpec((128, ED), lambda i: (0, 0)),
            pl.BlockSpec((1, ED), lambda i: (0, 0)),
        ],
        out_specs=[pl.BlockSpec((BO, DIN), lambda i: (i, 0)),
                   pl.BlockSpec((BO, DIN), lambda i: (i, 0))],
        out_shape=[jax.ShapeDtypeStruct((O, DIN), jnp.float32),
                   jax.ShapeDtypeStruct((O, DIN), jnp.float32)],
    )(objs.reshape(nb, 1, BO), boxes_pad, shapes_gt,
      emb_b_pad, emb_s_pad, wb_pad, bb, ws, bs)


def _embed_pred_body(p_ref, eb_ref, es_ref, pvb_ref, pvs_ref):
    idx = p_ref[0, 0, :]
    oh = (idx[:, None] == lax.broadcasted_iota(jnp.int32, (1, 128), 1)
          ).astype(jnp.float32)
    pvb_ref[...] = _dot(oh, eb_ref[...])
    pvs_ref[...] = _dot(oh, es_ref[...])


def _embed_preds(p, emb_b_pad, emb_s_pad):
    nb = TP // BT
    return pl.pallas_call(
        _embed_pred_body,
        grid=(nb,),
        in_specs=[
            pl.BlockSpec((1, 1, BT), lambda i: (i, 0, 0)),
            pl.BlockSpec((128, DIN), lambda i: (0, 0)),
            pl.BlockSpec((128, DIN), lambda i: (0, 0)),
        ],
        out_specs=[pl.BlockSpec((BT, DIN), lambda i: (i, 0)),
                   pl.BlockSpec((BT, DIN), lambda i: (i, 0))],
        out_shape=[jax.ShapeDtypeStruct((TP, DIN), jnp.float32),
                   jax.ShapeDtypeStruct((TP, DIN), jnp.float32)],
    )(p.reshape(nb, 1, BT), emb_b_pad, emb_s_pad)


def _stage_a_body(x_ref, ws_ref, wo_ref, a_ref, c_ref):
    x = x_ref[...]
    a_ref[...] = _dot(x, ws_ref[...])
    c_ref[...] = _dot(x, wo_ref[...])


def _stage_a(x, w1s, w1o):
    d = x.shape[1]
    nb = O // BO
    return pl.pallas_call(
        _stage_a_body,
        grid=(nb,),
        in_specs=[
            pl.BlockSpec((BO, d), lambda i: (i, 0)),
            pl.BlockSpec((d, H), lambda i: (0, 0)),
            pl.BlockSpec((d, H), lambda i: (0, 0)),
        ],
        out_specs=[pl.BlockSpec((BO, H), lambda i: (i, 0)),
                   pl.BlockSpec((BO, H), lambda i: (i, 0))],
        out_shape=[jax.ShapeDtypeStruct((O, H), jnp.float32),
                   jax.ShapeDtypeStruct((O, H), jnp.float32)],
    )(x, w1s, w1o)


def _stage_b_body(as_ref, cs_ref, pv_ref, w1p_ref, b1_ref, w2_ref, b2_ref,
                  ns_ref, np_ref, no_ref, *, dout):
    h = as_ref[...] + cs_ref[...] + _dot(pv_ref[...], w1p_ref[...]) + b1_ref[...]
    h = jnp.maximum(h, 0.0)
    nt = jnp.maximum(_dot(h, w2_ref[...]) + b2_ref[...], 0.0)
    ns_ref[...] = nt[:, :H]
    np_ref[...] = nt[:, H:H + dout]
    no_ref[...] = nt[:, H + dout:]


def _stage_b(As, Cs, pv, w1p, b1, w2, b2, dout):
    d = pv.shape[1]
    d2 = 2 * H + dout
    nb = TP // BT
    return pl.pallas_call(
        functools.partial(_stage_b_body, dout=dout),
        grid=(nb,),
        in_specs=[
            pl.BlockSpec((BT, H), lambda i: (i, 0)),
            pl.BlockSpec((BT, H), lambda i: (i, 0)),
            pl.BlockSpec((BT, d), lambda i: (i, 0)),
            pl.BlockSpec((d, H), lambda i: (0, 0)),
            pl.BlockSpec((1, H), lambda i: (0, 0)),
            pl.BlockSpec((H, d2), lambda i: (0, 0)),
            pl.BlockSpec((1, d2), lambda i: (0, 0)),
        ],
        out_specs=[pl.BlockSpec((BT, H), lambda i: (i, 0)),
                   pl.BlockSpec((BT, dout), lambda i: (i, 0)),
                   pl.BlockSpec((BT, H), lambda i: (i, 0))],
        out_shape=[jax.ShapeDtypeStruct((TP, H), jnp.float32),
                   jax.ShapeDtypeStruct((TP, dout), jnp.float32),
                   jax.ShapeDtypeStruct((TP, H), jnp.float32)],
    )(As, Cs, pv, w1p, b1, w2, b2)


def _stage_bd_body(os_ref, oo_ref, pv_ref, w1_ref, b1_ref, w2_ref, b2_ref,
                   ns_ref, np_ref, no_ref, *, dout):
    x = jnp.concatenate([os_ref[...], pv_ref[...], oo_ref[...]], axis=1)
    h = jnp.maximum(_dot(x, w1_ref[...]) + b1_ref[...], 0.0)
    nt = jnp.maximum(_dot(h, w2_ref[...]) + b2_ref[...], 0.0)
    ns_ref[...] = nt[:, :H]
    np_ref[...] = nt[:, H:H + dout]
    no_ref[...] = nt[:, H + dout:]


def _stage_bd(os_, oo, pv, w1, b1, w2, b2, dout):
    d = pv.shape[1]
    d2 = 2 * H + dout
    nb = TP // BT
    return pl.pallas_call(
        functools.partial(_stage_bd_body, dout=dout),
        grid=(nb,),
        in_specs=[
            pl.BlockSpec((BT, d), lambda i: (i, 0)),
            pl.BlockSpec((BT, d), lambda i: (i, 0)),
            pl.BlockSpec((BT, d), lambda i: (i, 0)),
            pl.BlockSpec((3 * d, H), lambda i: (0, 0)),
            pl.BlockSpec((1, H), lambda i: (0, 0)),
            pl.BlockSpec((H, d2), lambda i: (0, 0)),
            pl.BlockSpec((1, d2), lambda i: (0, 0)),
        ],
        out_specs=[pl.BlockSpec((BT, H), lambda i: (i, 0)),
                   pl.BlockSpec((BT, dout), lambda i: (i, 0)),
                   pl.BlockSpec((BT, H), lambda i: (i, 0))],
        out_shape=[jax.ShapeDtypeStruct((TP, H), jnp.float32),
                   jax.ShapeDtypeStruct((TP, dout), jnp.float32),
                   jax.ShapeDtypeStruct((TP, H), jnp.float32)],
    )(os_, oo, pv, w1, b1, w2, b2)


def _stage_c_body(pool_ref, cnt_ref, w3_ref, b3_ref, w4_ref, b4_ref, o_ref):
    x = pool_ref[...] / jnp.maximum(cnt_ref[...][:, :1], 1.0)
    h = jnp.maximum(_dot(x, w3_ref[...]) + b3_ref[...], 0.0)
    o_ref[...] = jnp.maximum(_dot(h, w4_ref[...]) + b4_ref[...], 0.0)


def _stage_c(pooled, counts_tab, w3, b3, w4, b4, dout):
    nb = O // BO
    return pl.pallas_call(
        _stage_c_body,
        grid=(nb,),
        in_specs=[
            pl.BlockSpec((BO, H), lambda i: (i, 0)),
            pl.BlockSpec((BO, H), lambda i: (i, 0)),
            pl.BlockSpec((H, H), lambda i: (0, 0)),
            pl.BlockSpec((1, H), lambda i: (0, 0)),
            pl.BlockSpec((H, dout), lambda i: (0, 0)),
            pl.BlockSpec((1, dout), lambda i: (0, 0)),
        ],
        out_specs=pl.BlockSpec((BO, dout), lambda i: (i, 0)),
        out_shape=jax.ShapeDtypeStruct((O, dout), jnp.float32),
    )(pooled, counts_tab, w3, b3, w4, b4)


def _stage_ca_body(pool_ref, cnt_ref, w3_ref, b3_ref, w4_ref, b4_ref,
                   ws_ref, wo_ref, o_ref, a_ref, c_ref):
    x = pool_ref[...] / jnp.maximum(cnt_ref[...][:, :1], 1.0)
    h = jnp.maximum(_dot(x, w3_ref[...]) + b3_ref[...], 0.0)
    y = jnp.maximum(_dot(h, w4_ref[...]) + b4_ref[...], 0.0)
    o_ref[...] = y
    a_ref[...] = _dot(y, ws_ref[...])
    c_ref[...] = _dot(y, wo_ref[...])


def _stage_ca(pooled, counts_tab, w3, b3, w4, b4, w1s, w1o, dout):
    nb = O // BO
    return pl.pallas_call(
        _stage_ca_body,
        grid=(nb,),
        in_specs=[
            pl.BlockSpec((BO, H), lambda i: (i, 0)),
            pl.BlockSpec((BO, H), lambda i: (i, 0)),
            pl.BlockSpec((H, H), lambda i: (0, 0)),
            pl.BlockSpec((1, H), lambda i: (0, 0)),
            pl.BlockSpec((H, dout), lambda i: (0, 0)),
            pl.BlockSpec((1, dout), lambda i: (0, 0)),
            pl.BlockSpec((dout, H), lambda i: (0, 0)),
            pl.BlockSpec((dout, H), lambda i: (0, 0)),
        ],
        out_specs=[pl.BlockSpec((BO, dout), lambda i: (i, 0)),
                   pl.BlockSpec((BO, H), lambda i: (i, 0)),
                   pl.BlockSpec((BO, H), lambda i: (i, 0))],
        out_shape=[jax.ShapeDtypeStruct((O, dout), jnp.float32),
                   jax.ShapeDtypeStruct((O, H), jnp.float32),
                   jax.ShapeDtypeStruct((O, H), jnp.float32)],
    )(pooled, counts_tab, w3, b3, w4, b4, w1s, w1o)


def _head_body(x_ref, w1_ref, b1_ref, w2_ref, b2_ref,
               wm_ref, bm_ref, wv_ref, bv_ref, mu_ref, lv_ref):
    h = jnp.maximum(_dot(x_ref[...], w1_ref[...]) + b1_ref[...], 0.0)
    hb = jnp.maximum(_dot(h, w2_ref[...]) + b2_ref[...], 0.0)
    mu_ref[...] = _dot(hb, wm_ref[...]) + bm_ref[...]
    lv_ref[...] = _dot(hb, wv_ref[...]) + bv_ref[...]


def _head(x, mv, mean, var):
    (w1, b1), (w2, b2) = mv
    (wm, bm), = mean
    (wv, bv), = var
    nb = O // BO
    return pl.pallas_call(
        _head_body,
        grid=(nb,),
        in_specs=[
            pl.BlockSpec((BO, DIN), lambda i: (i, 0)),
            pl.BlockSpec((DIN, H), lambda i: (0, 0)),
            pl.BlockSpec((1, H), lambda i: (0, 0)),
            pl.BlockSpec((H, DIN), lambda i: (0, 0)),
            pl.BlockSpec((1, DIN), lambda i: (0, 0)),
            pl.BlockSpec((DIN, ED), lambda i: (0, 0)),
            pl.BlockSpec((1, ED), lambda i: (0, 0)),
            pl.BlockSpec((DIN, ED), lambda i: (0, 0)),
            pl.BlockSpec((1, ED), lambda i: (0, 0)),
        ],
        out_specs=[pl.BlockSpec((BO, ED), lambda i: (i, 0)),
                   pl.BlockSpec((BO, ED), lambda i: (i, 0))],
        out_shape=[jax.ShapeDtypeStruct((O, ED), jnp.float32),
                   jax.ShapeDtypeStruct((O, ED), jnp.float32)],
    )(x, w1, b1.reshape(1, -1), w2, b2.reshape(1, -1),
      wm, bm.reshape(1, -1), wv, bv.reshape(1, -1))


def _layer_group(streams, idxs, counts_tab, zeros_hbm):
    sidx_g, oidx_g, sidx_sc, oidx_sc = idxs
    gathered = []
    for st in streams:
        if st["direct"]:
            gathered.append(_sc_gather(st["obj"], st["obj"], sidx_g, oidx_g))
        else:
            gathered.append(_sc_gather(st["ac"][0], st["ac"][1],
                                       sidx_g, oidx_g))
    edge_out = []
    for st, (ga, gc) in zip(streams, gathered):
        (w1, b1), (w2, b2) = st["gp"]["net1"]
        d = w1.shape[0] // 3
        if st["direct"]:
            edge_out.append(_stage_bd(ga, gc, st["pv"], w1,
                                      b1.reshape(1, -1), w2,
                                      b2.reshape(1, -1), st["dout"]))
        else:
            edge_out.append(_stage_b(ga, gc, st["pv"], w1[d:2 * d],
                                     b1.reshape(1, -1), w2,
                                     b2.reshape(1, -1), st["dout"]))
    pooled = [_sc_scatter(ns, no, sidx_sc, oidx_sc, zeros_hbm)
              for (ns, _, no) in edge_out]
    for st, (_, np_, __), pool in zip(streams, edge_out, pooled):
        (w3, b3), (w4, b4) = st["gp"]["net2"]
        st["pv"] = np_
        if st["direct"] or st["gp_next"] is None:
            st["obj"] = _stage_c(pool, counts_tab, w3, b3.reshape(1, -1),
                                 w4, b4.reshape(1, -1), st["dout"])
            st["ac"] = None
        else:
            w1n = st["gp_next"]["net1"][0][0]
            dn = w1n.shape[0] // 3
            st["obj"], a2, c2 = _stage_ca(
                pool, counts_tab, w3, b3.reshape(1, -1),
                w4, b4.reshape(1, -1), w1n[:dn], w1n[2 * dn:], st["dout"])
            st["ac"] = (a2, c2)
    return streams


def _prime(layers, obj_vecs):
    w1 = layers[0]["net1"][0][0]
    d = w1.shape[0] // 3
    return _stage_a(obj_vecs, w1[:d], w1[2 * d:])


def kernel(boxes_gt, shapes_gt, attributes, params, objs, triples):
    s = triples[:, 0]
    p = triples[:, 1]
    o = triples[:, 2]

    boxes_pad = jnp.pad(boxes_gt, ((0, 0), (0, 128 - boxes_gt.shape[1])))
    wb, bb = params["box_lin"][0]
    wb_pad = jnp.pad(wb, ((0, 128 - wb.shape[0]), (0, 0)))
    ws, bs = params["shape_lin"][0]
    eb_pad = jnp.pad(params["emb_obj_box"], ((0, 128 - NUM_OBJS - 1), (0, 0)))
    es_pad = jnp.pad(params["emb_obj_shape"], ((0, 128 - NUM_OBJS - 1), (0, 0)))
    peb_pad = jnp.pad(params["emb_pred_box"], ((0, 128 - NUM_PREDS), (0, 0)))
    pes_pad = jnp.pad(params["emb_pred_shape"], ((0, 128 - NUM_PREDS), (0, 0)))

    ovb, ovs = _embed_objs(objs, boxes_pad, shapes_gt, eb_pad, es_pad,
                           wb_pad, bb.reshape(1, -1), ws, bs.reshape(1, -1))
    p_pad = jnp.concatenate([p, jnp.zeros((TP - T,), p.dtype)])
    pvb, pvs = _embed_preds(p_pad, peb_pad, pes_pad)

    s32 = s.astype(jnp.int32)
    o32 = o.astype(jnp.int32)
    gpad = jnp.zeros((TP - T,), jnp.int32)
    spad = jnp.full((TP - T,), DUMP, jnp.int32)
    def _chunk_layout(idx, nblk, nch, nchp, cw):
        arr = jnp.zeros((nblk, nchp, cw), jnp.int32)
        arr = arr.at[:, :nch].set(idx.reshape(nblk, nch, cw))
        return arr.reshape(nblk * nchp, cw)

    sidx_g = _chunk_layout(jnp.concatenate([s32, gpad]), NW, GCH, GCHP, GCW)
    oidx_g = _chunk_layout(jnp.concatenate([o32, gpad]), NW, GCH, GCHP, GCW)
    sidx_sc = _chunk_layout(jnp.concatenate([s32, spad]), NS, SCH, SCHP, 128)
    oidx_sc = _chunk_layout(jnp.concatenate([o32, spad]), NS, SCH, SCHP, 128)
    idxs = (sidx_g, oidx_g, sidx_sc, oidx_sc)
    zeros_hbm = jnp.zeros((OP, 128), jnp.float32)
    ones_tp = jnp.ones((TP, H), jnp.float32)
    counts_tab = _sc_scatter(ones_tp, ones_tp, sidx_sc, oidx_sc, zeros_hbm)

    lb = params["gconv_ec_box"]
    ls = params["gconv_ec_shape"]
    sb = {"direct": True, "ac": None, "pv": pvb, "dout": DIN, "obj": ovb}
    ss = {"direct": True, "ac": None, "pv": pvs, "dout": DIN, "obj": ovs}
    for i in range(len(lb)):
        sb["gp"], ss["gp"] = lb[i], ls[i]
        sb["gp_next"] = lb[i + 1] if i + 1 < len(lb) else None
        ss["gp_next"] = ls[i + 1] if i + 1 < len(ls) else None
        sb, ss = _layer_group([sb, ss], idxs, counts_tab, zeros_hbm)
    ovb, pvb = sb["obj"], sb["pv"]
    ovs, pvs = ss["obj"], ss["pv"]

    ov = jnp.concatenate([ovb, ovs], axis=1)
    pv = jnp.concatenate([pvb, pvs], axis=1)
    lsh = params["gconv_shared"]
    sh = {"direct": False, "ac": _prime(lsh, ov), "pv": pv, "dout": H,
          "obj": None}
    for i in range(len(lsh)):
        sh["gp"] = lsh[i]
        sh["gp_next"] = lsh[i + 1] if i + 1 < len(lsh) else None
        (sh,) = _layer_group([sh], idxs, counts_tab, zeros_hbm)
    ov = sh["obj"]

    mu_box, lv_box = _head(ov[:, :DIN], params["box_mean_var"],
                           params["box_mean"], params["box_var"])
    mu_shape, lv_shape = _head(ov[:, DIN:], params["shape_mean_var"],
                               params["shape_mean"], params["shape_var"])
    return (mu_box, lv_box, mu_shape, lv_shape)
